# Initial kernel scaffold; baseline (speedup 1.0000x reference)
import functools

import jax
import jax.numpy as jnp
from jax import lax
from jax.experimental import pallas as pl
from jax.experimental.pallas import tpu as pltpu
from jax.experimental.pallas import tpu_sc as plsc

NC = 2
NS = 16
NW = NC * NS
LANES = 16
CHUNK = 128
BM = 512


def _sc_degree(row_idx, col_idx, npad):
    k_chunks = row_idx.shape[1]
    rows_per_tile = npad // NS
    mesh = plsc.VectorSubcoreMesh(core_axis_name="c", subcore_axis_name="s")

    @functools.partial(
        pl.kernel,
        out_type=(
            jax.ShapeDtypeStruct((NC, npad), jnp.float32),
            jax.ShapeDtypeStruct((NC, npad), jnp.float32),
        ),
        mesh=mesh,
        scratch_types=[
            pltpu.VMEM((k_chunks, CHUNK), jnp.int32),
            pltpu.VMEM((k_chunks, CHUNK), jnp.int32),
            pltpu.VMEM((CHUNK,), jnp.float32),
            pltpu.VMEM((CHUNK,), jnp.float32),
            pltpu.VMEM((rows_per_tile,), jnp.float32),
            pltpu.VMEM_SHARED((npad,), jnp.float32),
            pltpu.VMEM_SHARED((npad,), jnp.float32),
        ],
    )
    def deg_kernel(row_hbm, col_hbm, deg_out, self_out,
                   idxr, idxc, ones_v, sval, zb, deg_s, self_s):
        c = lax.axis_index("c")
        s = lax.axis_index("s")
        wid = c * NS + s
        base = s * rows_per_tile

        zeros16 = jnp.zeros((LANES,), jnp.float32)
        ones16 = jnp.ones((LANES,), jnp.float32)

        def zb_body(i, _):
            zb[pl.ds(i * LANES, LANES)] = zeros16
            return 0
        lax.fori_loop(0, rows_per_tile // LANES, zb_body, 0)

        for j in range(CHUNK // LANES):
            ones_v[pl.ds(j * LANES, LANES)] = ones16

        pltpu.sync_copy(zb, deg_s.at[pl.ds(base, rows_per_tile)])
        pltpu.sync_copy(zb, self_s.at[pl.ds(base, rows_per_tile)])
        pltpu.sync_copy(row_hbm.at[wid], idxr)
        pltpu.sync_copy(col_hbm.at[wid], idxc)
        plsc.subcore_barrier()

        def body(k, _):
            for j in range(CHUNK // LANES):
                rv = idxr[k, pl.ds(j * LANES, LANES)]
                cv = idxc[k, pl.ds(j * LANES, LANES)]
                sval[pl.ds(j * LANES, LANES)] = jnp.where(
                    rv == cv, 1.0, 0.0).astype(jnp.float32)
            pltpu.sync_copy(ones_v, deg_s.at[idxr.at[k]], add=True)
            pltpu.sync_copy(sval, self_s.at[idxr.at[k]], add=True)
            return 0
        lax.fori_loop(0, k_chunks, body, 0)

        plsc.subcore_barrier()
        pltpu.sync_copy(deg_s.at[pl.ds(base, rows_per_tile)],
                        deg_out.at[c, pl.ds(base, rows_per_tile)])
        pltpu.sync_copy(self_s.at[pl.ds(base, rows_per_tile)],
                        self_out.at[c, pl.ds(base, rows_per_tile)])

    return deg_kernel(row_idx, col_idx)


def _sc_segment_sum(y, row_idx, col_idx, npad, d):
    k_chunks = row_idx.shape[1]
    rows_per_tile = npad // NS
    mesh = plsc.VectorSubcoreMesh(core_axis_name="c", subcore_axis_name="s")

    @functools.partial(
        pl.kernel,
        out_type=jax.ShapeDtypeStruct((NC, npad, d), jnp.float32),
        mesh=mesh,
        scratch_types=[
            pltpu.VMEM((k_chunks, CHUNK), jnp.int32),
            pltpu.VMEM((k_chunks, CHUNK), jnp.int32),
            pltpu.VMEM((CHUNK, d), jnp.float32),
            pltpu.VMEM((CHUNK, d), jnp.float32),
            pltpu.VMEM_SHARED((npad, d), jnp.float32),
            pltpu.SemaphoreType.DMA,
        ],
    )
    def seg_kernel(y_hbm, row_hbm, col_hbm, s_out,
                   idxr, idxc, rows, zb, acc_s, sem):
        c = lax.axis_index("c")
        s = lax.axis_index("s")
        wid = c * NS + s
        base = s * rows_per_tile

        zeros16 = jnp.zeros((LANES,), jnp.float32)

        def zb_body(i, _):
            r = i // (d // LANES)
            col0 = (i % (d // LANES)) * LANES
            zb[r, pl.ds(col0, LANES)] = zeros16
            return 0
        lax.fori_loop(0, CHUNK * d // LANES, zb_body, 0)

        for r in range(rows_per_tile // CHUNK):
            pltpu.sync_copy(zb, acc_s.at[pl.ds(base + r * CHUNK, CHUNK)])
        pltpu.sync_copy(row_hbm.at[wid], idxr)
        pltpu.sync_copy(col_hbm.at[wid], idxc)
        plsc.subcore_barrier()

        def body(k, _):
            pltpu.async_copy(y_hbm.at[idxc.at[k]], rows, sem).wait()
            pltpu.sync_copy(rows, acc_s.at[idxr.at[k]], add=True)
            return 0
        lax.fori_loop(0, k_chunks, body, 0)

        plsc.subcore_barrier()
        pltpu.sync_copy(acc_s.at[pl.ds(base, rows_per_tile)],
                        s_out.at[c, pl.ds(base, rows_per_tile)])

    return seg_kernel(y, row_idx, col_idx)


def _tc_transform(x_pad, w, deg_t, npad, d):
    def body(x_ref, w_ref, deg_ref, y_ref):
        deg = jnp.sum(deg_ref[...], axis=1, keepdims=True)
        dinv = jnp.where(deg > 0, lax.rsqrt(deg), 0.0)
        y_ref[...] = dinv * jnp.dot(x_ref[...], w_ref[...],
                                    preferred_element_type=jnp.float32)

    return pl.pallas_call(
        body,
        grid=(npad // BM,),
        in_specs=[
            pl.BlockSpec((BM, d), lambda i: (i, 0)),
            pl.BlockSpec((d, d), lambda i: (0, 0)),
            pl.BlockSpec((BM, NC), lambda i: (i, 0)),
        ],
        out_specs=pl.BlockSpec((BM, d), lambda i: (i, 0)),
        out_shape=jax.ShapeDtypeStruct((npad, d), jnp.float32),
    )(x_pad, w, deg_t)


def _tc_final(s_parts, y, deg_t, self_t, b2, npad, d):
    def body(s_ref, y_ref, deg_ref, self_ref, b_ref, o_ref):
        deg = jnp.sum(deg_ref[...], axis=1, keepdims=True)
        dinv = jnp.where(deg > 0, lax.rsqrt(deg), 0.0)
        selfc = jnp.sum(self_ref[...], axis=1, keepdims=True)
        total = s_ref[0] + s_ref[1]
        o_ref[...] = dinv * total + (1.0 - selfc) * dinv * y_ref[...] + b_ref[...]

    return pl.pallas_call(
        body,
        grid=(npad // BM,),
        in_specs=[
            pl.BlockSpec((NC, BM, d), lambda i: (0, i, 0)),
            pl.BlockSpec((BM, d), lambda i: (i, 0)),
            pl.BlockSpec((BM, NC), lambda i: (i, 0)),
            pl.BlockSpec((BM, NC), lambda i: (i, 0)),
            pl.BlockSpec((1, d), lambda i: (0, 0)),
        ],
        out_specs=pl.BlockSpec((BM, d), lambda i: (i, 0)),
        out_shape=jax.ShapeDtypeStruct((npad, d), jnp.float32),
    )(s_parts, y, deg_t, self_t, b2)


def kernel(x, edge_index, W, b):
    n, d = x.shape
    e = edge_index.shape[1]

    x = x.astype(jnp.float32)
    W = W.astype(jnp.float32)
    b = b.astype(jnp.float32)

    npad = -(-(n + 1) // (NS * CHUNK)) * (NS * CHUNK)
    k_chunks = -(-e // (NW * CHUNK))
    epad = NW * CHUNK * k_chunks

    row = edge_index[0].astype(jnp.int32)
    col = edge_index[1].astype(jnp.int32)
    pad_idx = jnp.full((epad - e,), n, dtype=jnp.int32)
    row_p = jnp.concatenate([row, pad_idx]).reshape(NW, k_chunks, CHUNK)
    col_p = jnp.concatenate([col, pad_idx]).reshape(NW, k_chunks, CHUNK)

    x_pad = jnp.pad(x, ((0, npad - n), (0, 0)))

    deg_parts, self_parts = _sc_degree(row_p, col_p, npad)
    deg_t = deg_parts.T
    self_t = self_parts.T

    y = _tc_transform(x_pad, W, deg_t, npad, d)
    s_parts = _sc_segment_sum(y, row_p, col_p, npad, d)
    out = _tc_final(s_parts, y, deg_t, self_t, b.reshape(1, d), npad, d)
    return out[:n]


# speedup vs baseline: 14.2987x; 14.2987x over previous
import functools

import jax
import jax.numpy as jnp
from jax import lax
from jax.experimental import pallas as pl
from jax.experimental.pallas import tpu as pltpu
from jax.experimental.pallas import tpu_sc as plsc

NC = 2
NS = 16
NW = NC * NS
LANES = 16
CHUNK = 128
BM = 512
SHIFT = 14


def _i0():
    return jnp.int32(0)


def _sc_degree(packed_idx, npad):
    k_chunks = packed_idx.shape[1]
    rows_per_tile = npad // NS
    mesh = plsc.VectorSubcoreMesh(core_axis_name="c", subcore_axis_name="s")

    @functools.partial(
        pl.kernel,
        out_type=(
            jax.ShapeDtypeStruct((NC, npad), jnp.float32),
            jax.ShapeDtypeStruct((NC, npad), jnp.float32),
        ),
        mesh=mesh,
        scratch_types=[
            pltpu.VMEM((k_chunks, CHUNK), jnp.int32),
            pltpu.VMEM((1, CHUNK), jnp.int32),
            pltpu.VMEM((CHUNK,), jnp.float32),
            pltpu.VMEM((CHUNK,), jnp.float32),
            pltpu.VMEM((rows_per_tile,), jnp.float32),
            pltpu.VMEM_SHARED((npad,), jnp.float32),
            pltpu.VMEM_SHARED((npad,), jnp.float32),
        ],
    )
    def deg_kernel(pk_hbm, deg_out, self_out,
                   pk, idxr, ones_v, sval, zb, deg_s, self_s):
        c = lax.axis_index("c")
        s = lax.axis_index("s")
        wid = c * NS + s
        base = s * rows_per_tile

        zeros16 = jnp.zeros((LANES,), jnp.float32)
        ones16 = jnp.ones((LANES,), jnp.float32)

        def zb_body(i, _):
            zb[pl.ds(i * LANES, LANES)] = zeros16
            return _
        lax.fori_loop(jnp.int32(0), jnp.int32(rows_per_tile // LANES), zb_body, jnp.int32(0))

        for j in range(CHUNK // LANES):
            ones_v[pl.ds(j * LANES, LANES)] = ones16

        pltpu.sync_copy(zb, deg_s.at[pl.ds(base, rows_per_tile)])
        pltpu.sync_copy(zb, self_s.at[pl.ds(base, rows_per_tile)])
        pltpu.sync_copy(pk_hbm.at[wid], pk)
        plsc.subcore_barrier()

        def body(k, _):
            for j in range(CHUNK // LANES):
                pv = pk[k, pl.ds(j * LANES, LANES)]
                rv = lax.shift_right_logical(pv, jnp.int32(SHIFT))
                cv = lax.bitwise_and(pv, jnp.int32((1 << SHIFT) - 1))
                idxr[0, pl.ds(j * LANES, LANES)] = rv
                eq = 1 - jnp.minimum(jnp.abs(rv - cv), 1)
                sval[pl.ds(j * LANES, LANES)] = eq.astype(jnp.float32)
            pltpu.sync_copy(ones_v, deg_s.at[idxr.at[jnp.int32(0)]], add=True)
            pltpu.sync_copy(sval, self_s.at[idxr.at[jnp.int32(0)]], add=True)
            return _
        lax.fori_loop(jnp.int32(0), jnp.int32(k_chunks), body, jnp.int32(0))

        plsc.subcore_barrier()
        pltpu.sync_copy(deg_s.at[pl.ds(base, rows_per_tile)],
                        deg_out.at[c, pl.ds(base, rows_per_tile)])
        pltpu.sync_copy(self_s.at[pl.ds(base, rows_per_tile)],
                        self_out.at[c, pl.ds(base, rows_per_tile)])

    return deg_kernel(packed_idx)


def _sc_segment_sum(y, packed_idx, npad, d):
    k_chunks = packed_idx.shape[1]
    rows_per_tile = npad // NS
    mesh = plsc.VectorSubcoreMesh(core_axis_name="c", subcore_axis_name="s")

    @functools.partial(
        pl.kernel,
        out_type=jax.ShapeDtypeStruct((NC, npad, d), jnp.float32),
        mesh=mesh,
        scratch_types=[
            pltpu.VMEM((k_chunks, CHUNK), jnp.int32),
            pltpu.VMEM((8, CHUNK), jnp.int32),
            pltpu.VMEM((8, CHUNK), jnp.int32),
            pltpu.VMEM((CHUNK, d), jnp.float32),
            pltpu.VMEM((8, d), jnp.float32),
            pltpu.VMEM_SHARED((npad, d), jnp.float32),
            pltpu.SemaphoreType.DMA,
        ],
    )
    def seg_kernel(y_hbm, pk_hbm, s_out,
                   pk, idxr, idxc, rows, zb, acc_s, sem):
        c = lax.axis_index("c")
        s = lax.axis_index("s")
        wid = c * NS + s
        base = s * rows_per_tile

        zeros16 = jnp.zeros((LANES,), jnp.float32)

        def zb_body(i, _):
            r = i // (d // LANES)
            col0 = (i % (d // LANES)) * LANES
            zb[r, pl.ds(col0, LANES)] = zeros16
            return _
        lax.fori_loop(jnp.int32(0), jnp.int32(8 * d // LANES), zb_body, jnp.int32(0))

        def zacc_body(i, _):
            pltpu.sync_copy(zb, acc_s.at[pl.ds(base + i * 8, 8)])
            return _
        lax.fori_loop(jnp.int32(0), jnp.int32(rows_per_tile // 8), zacc_body, jnp.int32(0))
        pltpu.sync_copy(pk_hbm.at[wid], pk)
        plsc.subcore_barrier()

        zero_i = jnp.int32(0)

        def body(k, _):
            for j in range(CHUNK // LANES):
                pv = pk[k, pl.ds(j * LANES, LANES)]
                idxr[0, pl.ds(j * LANES, LANES)] = lax.shift_right_logical(
                    pv, jnp.int32(SHIFT))
                idxc[0, pl.ds(j * LANES, LANES)] = lax.bitwise_and(
                    pv, jnp.int32((1 << SHIFT) - 1))
            pltpu.async_copy(y_hbm.at[idxc.at[zero_i]], rows, sem).wait()
            pltpu.sync_copy(rows, acc_s.at[idxr.at[zero_i]], add=True)
            return _
        lax.fori_loop(jnp.int32(0), jnp.int32(k_chunks), body, jnp.int32(0))

        plsc.subcore_barrier()
        pltpu.sync_copy(acc_s.at[pl.ds(base, rows_per_tile)],
                        s_out.at[c, pl.ds(base, rows_per_tile)])

    return seg_kernel(y, packed_idx)


def _tc_transform(x_pad, w, deg_t, npad, d):
    def body(x_ref, w_ref, deg_ref, y_ref):
        deg = jnp.sum(deg_ref[...], axis=1, keepdims=True)
        dinv = jnp.where(deg > 0, lax.rsqrt(deg), 0.0)
        y_ref[...] = dinv * jnp.dot(x_ref[...], w_ref[...],
                                    preferred_element_type=jnp.float32)

    return pl.pallas_call(
        body,
        grid=(npad // BM,),
        in_specs=[
            pl.BlockSpec((BM, d), lambda i: (i, _i0())),
            pl.BlockSpec((d, d), lambda i: (_i0(), _i0())),
            pl.BlockSpec((BM, NC), lambda i: (i, _i0())),
        ],
        out_specs=pl.BlockSpec((BM, d), lambda i: (i, _i0())),
        out_shape=jax.ShapeDtypeStruct((npad, d), jnp.float32),
    )(x_pad, w, deg_t)


def _tc_final(s_parts, y, deg_t, self_t, b2, npad, d):
    def body(s_ref, y_ref, deg_ref, self_ref, b_ref, o_ref):
        deg = jnp.sum(deg_ref[...], axis=1, keepdims=True)
        dinv = jnp.where(deg > 0, lax.rsqrt(deg), 0.0)
        selfc = jnp.sum(self_ref[...], axis=1, keepdims=True)
        total = s_ref[0] + s_ref[1]
        o_ref[...] = dinv * total + (1.0 - selfc) * dinv * y_ref[...] + b_ref[...]

    return pl.pallas_call(
        body,
        grid=(npad // BM,),
        in_specs=[
            pl.BlockSpec((NC, BM, d), lambda i: (_i0(), i, _i0())),
            pl.BlockSpec((BM, d), lambda i: (i, _i0())),
            pl.BlockSpec((BM, NC), lambda i: (i, _i0())),
            pl.BlockSpec((BM, NC), lambda i: (i, _i0())),
            pl.BlockSpec((1, d), lambda i: (_i0(), _i0())),
        ],
        out_specs=pl.BlockSpec((BM, d), lambda i: (i, _i0())),
        out_shape=jax.ShapeDtypeStruct((npad, d), jnp.float32),
    )(s_parts, y, deg_t, self_t, b2)


def kernel(x, edge_index, W, b):
    n, d = x.shape
    e = edge_index.shape[1]

    x = x.astype(jnp.float32)
    W = W.astype(jnp.float32)
    b = b.astype(jnp.float32)

    npad = -(-(n + 1) // (NS * CHUNK)) * (NS * CHUNK)
    k_chunks = -(-e // (NW * CHUNK))
    epad = NW * CHUNK * k_chunks

    row = edge_index[0].astype(jnp.int32)
    col = edge_index[1].astype(jnp.int32)
    packed = jnp.bitwise_or(jnp.left_shift(row, SHIFT), col)
    pad_val = jnp.full((epad - e,), (n << SHIFT) | n, dtype=jnp.int32)
    packed = jnp.concatenate([packed, pad_val]).reshape(NW, k_chunks, CHUNK)

    x_pad = jnp.pad(x, ((0, npad - n), (0, 0)))

    deg_parts, self_parts = _sc_degree(packed, npad)
    deg_t = deg_parts.T
    self_t = self_parts.T

    y = _tc_transform(x_pad, W, deg_t, npad, d)
    s_parts = _sc_segment_sum(y, packed, npad, d)
    out = _tc_final(s_parts, y, deg_t, self_t, b.reshape(1, d), npad, d)
    return out[:n]
